# Initial kernel scaffold; baseline (speedup 1.0000x reference)
import jax
import jax.numpy as jnp
from jax.experimental import pallas as pl


def kernel(pos, batch):
    raise NotImplementedError("write your pallas kernel here")


# speedup vs baseline: 13.7900x; 13.7900x over previous
import functools

import jax
import jax.numpy as jnp
from jax.experimental import pallas as pl
from jax.experimental.pallas import tpu as pltpu

CUTOFF = 10.0
K = 32
R = 256
C = 512
POOL = 128 + C
INF = jnp.inf


def _tile_kernel(batch_smem, pos_row, batch_row, posT, batchT,
                 src_out, w_out, pool_val, pool_idx, bval, bidx, *, n):
    i = pl.program_id(0)
    r0 = i * R

    b_lo = batch_smem[r0]
    b_hi = batch_smem[r0 + R - 1]

    xr = pos_row[:, 0:1]
    yr = pos_row[:, 1:2]
    zr = pos_row[:, 2:3]
    brow = batch_row[:, 0:1]
    rowg = r0 + jax.lax.broadcasted_iota(jnp.int32, (R, 1), 0)

    lane_pool = jax.lax.broadcasted_iota(jnp.int32, (R, POOL), 1)
    lane_best = jax.lax.broadcasted_iota(jnp.int32, (R, 128), 1)
    lane_chunk = jax.lax.broadcasted_iota(jnp.int32, (R, C), 1)

    bval[...] = jnp.full((R, 128), INF, jnp.float32)
    bidx[...] = jnp.zeros((R, 128), jnp.int32)

    num_chunks = n // C
    for c in range(num_chunks):
        c0 = c * C
        c_blo = batch_smem[c0]
        c_bhi = batch_smem[c0 + C - 1]

        @pl.when((c_bhi >= b_lo) & (c_blo <= b_hi))
        def _merge(c0=c0):
            xc = posT[0:1, c0:c0 + C]
            yc = posT[1:2, c0:c0 + C]
            zc = posT[2:3, c0:c0 + C]
            dx = xr - xc
            dy = yr - yc
            dz = zr - zc
            d2 = (dx * dx + dy * dy) + dz * dz
            dist = jnp.sqrt(d2)
            bcol = batchT[0:1, c0:c0 + C]
            colg = c0 + lane_chunk
            ok = (brow == bcol) & (colg != rowg) & (dist <= CUTOFF)
            pool_val[:, 128:POOL] = jnp.where(ok, dist, INF)
            pool_idx[:, 128:POOL] = colg
            pool_val[:, 0:128] = bval[...]
            pool_idx[:, 0:128] = bidx[...]

            def _select(k, _):
                pv = pool_val[...]
                m = jnp.min(pv, axis=1, keepdims=True)
                eq = pv == m
                pos = jnp.min(jnp.where(eq, lane_pool, POOL), axis=1,
                              keepdims=True)
                selmask = lane_pool == pos
                sidx = jnp.max(jnp.where(selmask, pool_idx[...], -1), axis=1,
                               keepdims=True)
                pool_val[...] = jnp.where(selmask, INF, pv)
                kmask = lane_best == k
                bval[...] = jnp.where(kmask, m, bval[...])
                bidx[...] = jnp.where(kmask, sidx, bidx[...])
                return 0

            jax.lax.fori_loop(0, K, _select, 0)

    best_v = bval[:, 0:K]
    best_i = bidx[:, 0:K]
    finite = best_v < INF
    center = jnp.broadcast_to(rowg, (R, K))
    src_out[...] = jnp.where(finite, best_i, center)
    w_out[...] = jnp.where(finite, best_v, 0.0)


@jax.jit
def _radius_graph(pos, batch):
    n = pos.shape[0]
    batch = batch.astype(jnp.int32)
    posT = pos.T
    batchT = batch.reshape(1, n)
    batch2d = batch.reshape(n, 1)

    grid = (n // R,)
    src, w = pl.pallas_call(
        functools.partial(_tile_kernel, n=n),
        grid=grid,
        in_specs=[
            pl.BlockSpec(memory_space=pltpu.SMEM),
            pl.BlockSpec((R, 3), lambda i: (i, 0)),
            pl.BlockSpec((R, 1), lambda i: (i, 0)),
            pl.BlockSpec((3, n), lambda i: (0, 0)),
            pl.BlockSpec((1, n), lambda i: (0, 0)),
        ],
        out_specs=[
            pl.BlockSpec((R, K), lambda i: (i, 0)),
            pl.BlockSpec((R, K), lambda i: (i, 0)),
        ],
        out_shape=[
            jax.ShapeDtypeStruct((n, K), jnp.int32),
            jax.ShapeDtypeStruct((n, K), jnp.float32),
        ],
        scratch_shapes=[
            pltpu.VMEM((R, POOL), jnp.float32),
            pltpu.VMEM((R, POOL), jnp.int32),
            pltpu.VMEM((R, 128), jnp.float32),
            pltpu.VMEM((R, 128), jnp.int32),
        ],
    )(batch, pos, batch2d, posT, batchT)

    centers = jnp.broadcast_to(jnp.arange(n, dtype=jnp.int32)[:, None],
                               (n, K))
    edge_index = jnp.stack([src.reshape(-1), centers.reshape(-1)], axis=0)
    edge_weight = w.reshape(-1)
    return edge_index, edge_weight


def kernel(pos, batch):
    return _radius_graph(pos, batch)


# speedup vs baseline: 26.5095x; 1.9224x over previous
import functools

import jax
import jax.numpy as jnp
from jax.experimental import pallas as pl
from jax.experimental.pallas import tpu as pltpu

CUTOFF = 10.0
K = 32
R = 256
C = 512
INF = jnp.inf


def _tile_kernel(cfirst_smem, fast_smem, pos_row, batch_row, posT, batchT,
                 src_out, w_out, buf, bval, bidx, *, n):
    i = pl.program_id(0)
    r0 = i * R
    num_chunks = n // C

    xr = pos_row[:, 0:1]
    yr = pos_row[:, 1:2]
    zr = pos_row[:, 2:3]
    brow = batch_row[:, 0:1]
    rowg = r0 + jax.lax.broadcasted_iota(jnp.int32, (R, 1), 0)
    lane_chunk = jax.lax.broadcasted_iota(jnp.int32, (R, C), 1)
    lane_best = jax.lax.broadcasted_iota(jnp.int32, (R, 128), 1)

    def masked_dist(c0):
        xc = posT[0:1, c0:c0 + C]
        yc = posT[1:2, c0:c0 + C]
        zc = posT[2:3, c0:c0 + C]
        dx = xr - xc
        dy = yr - yc
        dz = zr - zc
        dist = jnp.sqrt((dx * dx + dy * dy) + dz * dz)
        bcol = batchT[0:1, c0:c0 + C]
        colg = c0 + lane_chunk
        ok = (brow == bcol) & (colg != rowg) & (dist <= CUTOFF)
        return jnp.where(ok, dist, INF)

    def select(base, w):
        iota = jax.lax.broadcasted_iota(jnp.int32, (R, w), 1)

        def body(k, _):
            bv = buf[:, 0:w]
            m = jnp.min(bv, axis=1, keepdims=True)
            pos = jnp.min(jnp.where(bv == m, iota, w), axis=1, keepdims=True)
            buf[:, 0:w] = jnp.where(iota == pos, INF, bv)
            kmask = lane_best == k
            bval[...] = jnp.where(kmask, m, bval[...])
            bidx[...] = jnp.where(kmask, base + pos, bidx[...])
            return 0

        jax.lax.fori_loop(0, K, body, 0)

    c1 = cfirst_smem[i]
    isfast = fast_smem[i] == 1

    @pl.when(isfast)
    def _fast():
        buf[:, C:2 * C] = jnp.full((R, C), INF, jnp.float32)
        for c in range(num_chunks):
            @pl.when(c1 == c)
            def _lo(c=c):
                buf[:, 0:C] = masked_dist(c * C)
            if c >= 1:
                @pl.when(c1 == c - 1)
                def _hi(c=c):
                    buf[:, C:2 * C] = masked_dist(c * C)
        select(c1 * C, 2 * C)

    @pl.when(jnp.logical_not(isfast))
    def _slow():
        for c in range(num_chunks):
            buf[:, c * C:(c + 1) * C] = masked_dist(c * C)
        select(0, n)

    best_v = bval[:, 0:K]
    best_i = bidx[:, 0:K]
    finite = best_v < INF
    center = jnp.broadcast_to(rowg, (R, K))
    src_out[...] = jnp.where(finite, best_i, center)
    w_out[...] = jnp.where(finite, best_v, 0.0)


@jax.jit
def _radius_graph(pos, batch):
    n = pos.shape[0]
    batch = batch.astype(jnp.int32)
    posT = pos.T
    batchT = batch.reshape(1, n)
    batch2d = batch.reshape(n, 1)

    tile_first = batch[::R]
    tile_last = batch[R - 1::R]
    win_start = jnp.searchsorted(batch, tile_first, side="left")
    win_end = jnp.searchsorted(batch, tile_last, side="right")
    c_first = (win_start // C).astype(jnp.int32)
    fast = (win_end <= (c_first + 2) * C).astype(jnp.int32)

    grid = (n // R,)
    src, w = pl.pallas_call(
        functools.partial(_tile_kernel, n=n),
        grid=grid,
        in_specs=[
            pl.BlockSpec(memory_space=pltpu.SMEM),
            pl.BlockSpec(memory_space=pltpu.SMEM),
            pl.BlockSpec((R, 3), lambda i: (i, 0)),
            pl.BlockSpec((R, 1), lambda i: (i, 0)),
            pl.BlockSpec((3, n), lambda i: (0, 0)),
            pl.BlockSpec((1, n), lambda i: (0, 0)),
        ],
        out_specs=[
            pl.BlockSpec((R, K), lambda i: (i, 0)),
            pl.BlockSpec((R, K), lambda i: (i, 0)),
        ],
        out_shape=[
            jax.ShapeDtypeStruct((n, K), jnp.int32),
            jax.ShapeDtypeStruct((n, K), jnp.float32),
        ],
        scratch_shapes=[
            pltpu.VMEM((R, n), jnp.float32),
            pltpu.VMEM((R, 128), jnp.float32),
            pltpu.VMEM((R, 128), jnp.int32),
        ],
    )(c_first, fast, pos, batch2d, posT, batchT)

    centers = jnp.broadcast_to(jnp.arange(n, dtype=jnp.int32)[:, None],
                               (n, K))
    edge_index = jnp.stack([src.reshape(-1), centers.reshape(-1)], axis=0)
    edge_weight = w.reshape(-1)
    return edge_index, edge_weight


def kernel(pos, batch):
    return _radius_graph(pos, batch)
